# baseline (speedup 1.0000x reference)
import functools

import jax
import jax.numpy as jnp
from jax import lax
from jax.experimental import pallas as pl
from jax.experimental.pallas import tpu as pltpu
from jax.experimental.pallas import tpu_sc as plsc

N = 10000
E = 320000
D = 128

NC = 2
NS = 16
NW = NC * NS
CH = 128
NCH = 80
GRP = 40
EPW = CH * NCH
EPAD = NW * EPW
NP = 10240
RPT = NP // NS
PAD_DST = NP - 8


_MESH = plsc.VectorSubcoreMesh(core_axis_name="c", subcore_axis_name="s")


def _sc_agg_body(x_hbm, src_hbm, dst_hbm, zrow_hbm,
                 agg_out, src_v, dst_v, rows0, rows1, agg_sh, sem0, sem1):
  c = lax.axis_index("c")
  s = lax.axis_index("s")
  wid = c * NS + s
  base = s * RPT

  pltpu.sync_copy(zrow_hbm, agg_sh.at[pl.ds(base, RPT)])
  plsc.subcore_barrier()

  half = GRP // 2
  for g in range(NCH // GRP):
    pltpu.sync_copy(src_hbm.at[wid].at[pl.ds(g * GRP, GRP)], src_v)
    pltpu.sync_copy(dst_hbm.at[wid].at[pl.ds(g * GRP, GRP)], dst_v)
    pltpu.async_copy(x_hbm.at[src_v.at[0]], rows0, sem0)
    pltpu.async_copy(x_hbm.at[src_v.at[1]], rows1, sem1)

    def pipe(i, carry):
      pltpu.make_async_copy(x_hbm.at[src_v.at[2 * i]], rows0, sem0).wait()
      pltpu.sync_copy(rows0, agg_sh.at[dst_v.at[2 * i]], add=True)

      @pl.when(i < half - 1)
      def _():
        pltpu.async_copy(x_hbm.at[src_v.at[2 * i + 2]], rows0, sem0)

      pltpu.make_async_copy(x_hbm.at[src_v.at[2 * i + 1]], rows1, sem1).wait()
      pltpu.sync_copy(rows1, agg_sh.at[dst_v.at[2 * i + 1]], add=True)

      @pl.when(i < half - 1)
      def _():
        pltpu.async_copy(x_hbm.at[src_v.at[2 * i + 3]], rows1, sem1)

      return carry

    lax.fori_loop(0, half, pipe, 0)
  plsc.subcore_barrier()

  pltpu.sync_copy(agg_sh.at[pl.ds(base, RPT)],
                  agg_out.at[c].at[pl.ds(base, RPT)])


_sc_agg = pl.kernel(
    _sc_agg_body,
    out_type=jax.ShapeDtypeStruct((NC, NP, D), jnp.float32),
    mesh=_MESH,
    scratch_types=[
        pltpu.VMEM((GRP, CH), jnp.int32),
        pltpu.VMEM((GRP, CH), jnp.int32),
        pltpu.VMEM((CH, D), jnp.float32),
        pltpu.VMEM((CH, D), jnp.float32),
        pltpu.VMEM_SHARED((NP, D), jnp.float32),
        pltpu.SemaphoreType.DMA,
        pltpu.SemaphoreType.DMA,
    ])


def _sc_cnt_body(dst_hbm, zrow_hbm, ones_hbm,
                 cnt_out, dst_v, ones_v, cnt_sh):
  c = lax.axis_index("c")
  s = lax.axis_index("s")
  wid = c * NS + s
  base = s * RPT

  pltpu.sync_copy(dst_hbm.at[wid], dst_v)
  pltpu.sync_copy(ones_hbm, ones_v)
  pltpu.sync_copy(zrow_hbm, cnt_sh.at[pl.ds(base, RPT)])
  plsc.subcore_barrier()

  def chunk(j, carry):
    pltpu.sync_copy(ones_v, cnt_sh.at[dst_v.at[j]], add=True)
    return carry

  lax.fori_loop(0, NCH, chunk, 0)
  plsc.subcore_barrier()

  pltpu.sync_copy(cnt_sh.at[pl.ds(base, RPT)],
                  cnt_out.at[c].at[pl.ds(base, RPT)])


_sc_cnt = pl.kernel(
    _sc_cnt_body,
    out_type=jax.ShapeDtypeStruct((NC, NP, D), jnp.float32),
    mesh=_MESH,
    scratch_types=[
        pltpu.VMEM((NCH, CH), jnp.int32),
        pltpu.VMEM((CH, D), jnp.float32),
        pltpu.VMEM_SHARED((NP, D), jnp.float32),
    ])

RB = 1000


def _tc_layer1(x_ref, a0_ref, a1_ref, c0_ref, c1_ref,
               w1lt_ref, b1l_ref, w1rt_ref, h1_ref):
  cnt = c0_ref[:, 0:1] + c1_ref[:, 0:1]
  inv = 1.0 / jnp.maximum(cnt, 1.0)
  mean = (a0_ref[...] + a1_ref[...]) * inv
  h1 = (jnp.dot(mean, w1lt_ref[...], preferred_element_type=jnp.float32)
        + b1l_ref[...]
        + jnp.dot(x_ref[...], w1rt_ref[...],
                  preferred_element_type=jnp.float32))
  h1_ref[...] = jnp.maximum(h1, 0.0)


def _tc_layer2(h1_ref, a0_ref, a1_ref, c0_ref, c1_ref,
               w2lt_ref, b2l_ref, w2rt_ref,
               wl1t_ref, wl2t_ref, blin_ref, out_ref):
  cnt = c0_ref[:, 0:1] + c1_ref[:, 0:1]
  inv = 1.0 / jnp.maximum(cnt, 1.0)
  mean = (a0_ref[...] + a1_ref[...]) * inv
  h1 = h1_ref[...]
  h2 = (jnp.dot(mean, w2lt_ref[...], preferred_element_type=jnp.float32)
        + b2l_ref[...]
        + jnp.dot(h1, w2rt_ref[...], preferred_element_type=jnp.float32))
  h2 = jnp.maximum(h2, 0.0)
  out = (jnp.dot(h1, wl1t_ref[...], preferred_element_type=jnp.float32)
         + jnp.dot(h2, wl2t_ref[...], preferred_element_type=jnp.float32)
         + blin_ref[...])
  out_ref[...] = jnp.maximum(out, 0.0)


def _row_spec(width):
  return pl.BlockSpec((RB, width), lambda i: (i, 0))


def _full_spec(shape):
  return pl.BlockSpec(shape, lambda i: tuple(0 for _ in shape))


def kernel(x, edge_index, W1l, b1l, W1r, W2l, b2l, W2r, Wlin, blin):
  src = jnp.concatenate(
      [edge_index[0].astype(jnp.int32),
       jnp.zeros((EPAD - E,), jnp.int32)]).reshape(NW, NCH, CH)
  dst = jnp.concatenate(
      [edge_index[1].astype(jnp.int32),
       jnp.full((EPAD - E,), PAD_DST, jnp.int32)]).reshape(NW, NCH, CH)
  zrow = jnp.zeros((RPT, D), jnp.float32)
  ones = jnp.ones((CH, D), jnp.float32)

  cnt = _sc_cnt(dst, zrow, ones)
  agg1 = _sc_agg(x, src, dst, zrow)
  c0 = cnt[0]
  c1 = cnt[1]

  grid = (N // RB,)
  h1 = pl.pallas_call(
      _tc_layer1,
      grid=grid,
      in_specs=[_row_spec(D), _row_spec(D), _row_spec(D),
                _row_spec(D), _row_spec(D),
                _full_spec((D, D)), _full_spec((D,)), _full_spec((D, D))],
      out_specs=_row_spec(D),
      out_shape=jax.ShapeDtypeStruct((N, D), jnp.float32),
  )(x, agg1[0], agg1[1], c0, c1, W1l.T, b1l, W1r.T)

  agg2 = _sc_agg(h1, src, dst, zrow)

  out = pl.pallas_call(
      _tc_layer2,
      grid=grid,
      in_specs=[_row_spec(D), _row_spec(D), _row_spec(D),
                _row_spec(D), _row_spec(D),
                _full_spec((D, D)), _full_spec((D,)), _full_spec((D, D)),
                _full_spec((D, D)), _full_spec((D, D)), _full_spec((D,))],
      out_specs=_row_spec(D),
      out_shape=jax.ShapeDtypeStruct((N, D), jnp.float32),
  )(h1, agg2[0], agg2[1], c0, c1, W2l.T, b2l, W2r.T,
    Wlin[:, :D].T, Wlin[:, D:].T, blin)
  return out


# speedup vs baseline: 1.0003x; 1.0003x over previous
import functools

import jax
import jax.numpy as jnp
from jax import lax
from jax.experimental import pallas as pl
from jax.experimental.pallas import tpu as pltpu
from jax.experimental.pallas import tpu_sc as plsc

N = 10000
E = 320000
D = 128

NC = 2
NS = 16
NW = NC * NS
CH = 128
NCH = 80
GRP = 40
EPW = CH * NCH
EPAD = NW * EPW
NP = 10240
RPT = NP // NS
PAD_DST = NP - 8


_MESH = plsc.VectorSubcoreMesh(core_axis_name="c", subcore_axis_name="s")


def _sc_agg_body(x_hbm, src_hbm, dst_hbm, zrow_hbm,
                 agg_out, src_v, dst_v, rows0, rows1, agg_sh, sem0, sem1):
  c = lax.axis_index("c")
  s = lax.axis_index("s")
  wid = c * NS + s
  base = s * RPT

  pltpu.sync_copy(zrow_hbm, agg_sh.at[pl.ds(base, RPT)])
  plsc.subcore_barrier()

  half = GRP // 2
  for g in range(NCH // GRP):
    pltpu.sync_copy(src_hbm.at[wid].at[pl.ds(g * GRP, GRP)], src_v)
    pltpu.sync_copy(dst_hbm.at[wid].at[pl.ds(g * GRP, GRP)], dst_v)
    pltpu.async_copy(x_hbm.at[src_v.at[0]], rows0, sem0)
    pltpu.async_copy(x_hbm.at[src_v.at[1]], rows1, sem1)

    def pipe(i, carry):
      pltpu.make_async_copy(zrow_hbm.at[pl.ds(0, CH)], rows0, sem0).wait()
      pltpu.sync_copy(rows0, agg_sh.at[dst_v.at[2 * i]], add=True)

      @pl.when(i < half - 1)
      def _():
        pltpu.async_copy(x_hbm.at[src_v.at[2 * i + 2]], rows0, sem0)

      pltpu.make_async_copy(zrow_hbm.at[pl.ds(0, CH)], rows1, sem1).wait()
      pltpu.sync_copy(rows1, agg_sh.at[dst_v.at[2 * i + 1]], add=True)

      @pl.when(i < half - 1)
      def _():
        pltpu.async_copy(x_hbm.at[src_v.at[2 * i + 3]], rows1, sem1)

      return carry

    lax.fori_loop(0, half, pipe, 0)
  plsc.subcore_barrier()

  pltpu.sync_copy(agg_sh.at[pl.ds(base, RPT)],
                  agg_out.at[c].at[pl.ds(base, RPT)])


_sc_agg = pl.kernel(
    _sc_agg_body,
    out_type=jax.ShapeDtypeStruct((NC, NP, D), jnp.float32),
    mesh=_MESH,
    scratch_types=[
        pltpu.VMEM((GRP, CH), jnp.int32),
        pltpu.VMEM((GRP, CH), jnp.int32),
        pltpu.VMEM((CH, D), jnp.float32),
        pltpu.VMEM((CH, D), jnp.float32),
        pltpu.VMEM_SHARED((NP, D), jnp.float32),
        pltpu.SemaphoreType.DMA,
        pltpu.SemaphoreType.DMA,
    ])


def _sc_cnt_body(dst_hbm, zrow_hbm, ones_hbm,
                 cnt_out, dst_v, ones_v, cnt_sh):
  c = lax.axis_index("c")
  s = lax.axis_index("s")
  wid = c * NS + s
  base = s * RPT

  pltpu.sync_copy(dst_hbm.at[wid], dst_v)
  pltpu.sync_copy(ones_hbm, ones_v)
  pltpu.sync_copy(zrow_hbm, cnt_sh.at[pl.ds(base, RPT)])
  plsc.subcore_barrier()

  def chunk(j, carry):
    pltpu.sync_copy(ones_v, cnt_sh.at[dst_v.at[j]], add=True)
    return carry

  lax.fori_loop(0, NCH, chunk, 0)
  plsc.subcore_barrier()

  pltpu.sync_copy(cnt_sh.at[pl.ds(base, RPT)],
                  cnt_out.at[c].at[pl.ds(base, RPT)])


_sc_cnt = pl.kernel(
    _sc_cnt_body,
    out_type=jax.ShapeDtypeStruct((NC, NP, D), jnp.float32),
    mesh=_MESH,
    scratch_types=[
        pltpu.VMEM((NCH, CH), jnp.int32),
        pltpu.VMEM((CH, D), jnp.float32),
        pltpu.VMEM_SHARED((NP, D), jnp.float32),
    ])

RB = 1000


def _tc_layer1(x_ref, a0_ref, a1_ref, c0_ref, c1_ref,
               w1lt_ref, b1l_ref, w1rt_ref, h1_ref):
  cnt = c0_ref[:, 0:1] + c1_ref[:, 0:1]
  inv = 1.0 / jnp.maximum(cnt, 1.0)
  mean = (a0_ref[...] + a1_ref[...]) * inv
  h1 = (jnp.dot(mean, w1lt_ref[...], preferred_element_type=jnp.float32)
        + b1l_ref[...]
        + jnp.dot(x_ref[...], w1rt_ref[...],
                  preferred_element_type=jnp.float32))
  h1_ref[...] = jnp.maximum(h1, 0.0)


def _tc_layer2(h1_ref, a0_ref, a1_ref, c0_ref, c1_ref,
               w2lt_ref, b2l_ref, w2rt_ref,
               wl1t_ref, wl2t_ref, blin_ref, out_ref):
  cnt = c0_ref[:, 0:1] + c1_ref[:, 0:1]
  inv = 1.0 / jnp.maximum(cnt, 1.0)
  mean = (a0_ref[...] + a1_ref[...]) * inv
  h1 = h1_ref[...]
  h2 = (jnp.dot(mean, w2lt_ref[...], preferred_element_type=jnp.float32)
        + b2l_ref[...]
        + jnp.dot(h1, w2rt_ref[...], preferred_element_type=jnp.float32))
  h2 = jnp.maximum(h2, 0.0)
  out = (jnp.dot(h1, wl1t_ref[...], preferred_element_type=jnp.float32)
         + jnp.dot(h2, wl2t_ref[...], preferred_element_type=jnp.float32)
         + blin_ref[...])
  out_ref[...] = jnp.maximum(out, 0.0)


def _row_spec(width):
  return pl.BlockSpec((RB, width), lambda i: (i, 0))


def _full_spec(shape):
  return pl.BlockSpec(shape, lambda i: tuple(0 for _ in shape))


def kernel(x, edge_index, W1l, b1l, W1r, W2l, b2l, W2r, Wlin, blin):
  src = jnp.concatenate(
      [edge_index[0].astype(jnp.int32),
       jnp.zeros((EPAD - E,), jnp.int32)]).reshape(NW, NCH, CH)
  dst = jnp.concatenate(
      [edge_index[1].astype(jnp.int32),
       jnp.full((EPAD - E,), PAD_DST, jnp.int32)]).reshape(NW, NCH, CH)
  zrow = jnp.zeros((RPT, D), jnp.float32)
  ones = jnp.ones((CH, D), jnp.float32)

  cnt = _sc_cnt(dst, zrow, ones)
  agg1 = _sc_agg(x, src, dst, zrow)
  c0 = cnt[0]
  c1 = cnt[1]

  grid = (N // RB,)
  h1 = pl.pallas_call(
      _tc_layer1,
      grid=grid,
      in_specs=[_row_spec(D), _row_spec(D), _row_spec(D),
                _row_spec(D), _row_spec(D),
                _full_spec((D, D)), _full_spec((D,)), _full_spec((D, D))],
      out_specs=_row_spec(D),
      out_shape=jax.ShapeDtypeStruct((N, D), jnp.float32),
  )(x, agg1[0], agg1[1], c0, c1, W1l.T, b1l, W1r.T)

  agg2 = _sc_agg(h1, src, dst, zrow)

  out = pl.pallas_call(
      _tc_layer2,
      grid=grid,
      in_specs=[_row_spec(D), _row_spec(D), _row_spec(D),
                _row_spec(D), _row_spec(D),
                _full_spec((D, D)), _full_spec((D,)), _full_spec((D, D)),
                _full_spec((D, D)), _full_spec((D, D)), _full_spec((D,))],
      out_specs=_row_spec(D),
      out_shape=jax.ShapeDtypeStruct((N, D), jnp.float32),
  )(h1, agg2[0], agg2[1], c0, c1, W2l.T, b2l, W2r.T,
    Wlin[:, :D].T, Wlin[:, D:].T, blin)
  return out


# speedup vs baseline: 1.3808x; 1.3804x over previous
import jax
import jax.numpy as jnp
from jax import lax
from jax.experimental import pallas as pl
from jax.experimental.pallas import tpu as pltpu
from jax.experimental.pallas import tpu_sc as plsc

N = 10000
E = 320000
D = 128

NC = 2
NS = 16
NW = NC * NS
CH = 128
NCH = 79
EPW = CH * NCH
EPAD = NW * EPW
NP = 10240
RPT = NP // NS
PAD_DST = NP - 8


_MESH = plsc.VectorSubcoreMesh(core_axis_name="c", subcore_axis_name="s")


def _make_sc_agg(with_cnt):
  def body(*refs):
    if with_cnt:
      (x_hbm, src_hbm, dst_hbm, zrow_hbm, ones_hbm,
       agg_out, cnt_out, src_v, dst_v, rows_v, agg_sh, sem) = refs
    else:
      (x_hbm, src_hbm, dst_hbm, zrow_hbm,
       agg_out, src_v, dst_v, rows_v, agg_sh, sem) = refs

    c = lax.axis_index("c")
    s = lax.axis_index("s")
    wid = c * NS + s
    base = s * RPT

    pltpu.sync_copy(src_hbm.at[wid], src_v)
    pltpu.sync_copy(dst_hbm.at[wid], dst_v)
    pltpu.sync_copy(zrow_hbm, agg_sh.at[pl.ds(base, RPT)])
    plsc.subcore_barrier()

    def chunk(j, carry):
      pltpu.async_copy(x_hbm.at[src_v.at[j]], rows_v, sem).wait()
      pltpu.sync_copy(rows_v, agg_sh.at[dst_v.at[j]], add=True)
      return carry

    lax.fori_loop(0, NCH, chunk, 0)
    plsc.subcore_barrier()

    pltpu.sync_copy(agg_sh.at[pl.ds(base, RPT)],
                    agg_out.at[c].at[pl.ds(base, RPT)])

    if with_cnt:
      pltpu.sync_copy(zrow_hbm, agg_sh.at[pl.ds(base, RPT)])
      pltpu.sync_copy(ones_hbm, rows_v)
      plsc.subcore_barrier()

      def chunk2(j, carry):
        pltpu.sync_copy(rows_v, agg_sh.at[dst_v.at[j]], add=True)
        return carry

      lax.fori_loop(0, NCH, chunk2, 0)
      plsc.subcore_barrier()
      pltpu.sync_copy(agg_sh.at[pl.ds(base, RPT)],
                      cnt_out.at[c].at[pl.ds(base, RPT)])

  out_type = [jax.ShapeDtypeStruct((NC, NP, D), jnp.float32)]
  if with_cnt:
    out_type.append(jax.ShapeDtypeStruct((NC, NP, D), jnp.float32))
  return pl.kernel(
      body,
      out_type=tuple(out_type) if with_cnt else out_type[0],
      mesh=_MESH,
      scratch_types=[
          pltpu.VMEM((NCH, CH), jnp.int32),
          pltpu.VMEM((NCH, CH), jnp.int32),
          pltpu.VMEM((CH, D), jnp.float32),
          pltpu.VMEM_SHARED((NP, D), jnp.float32),
          pltpu.SemaphoreType.DMA,
      ])


_sc_agg_cnt = _make_sc_agg(True)
_sc_agg = _make_sc_agg(False)

RB = 1000


def _tc_layer1(x_ref, a0_ref, a1_ref, c0_ref, c1_ref,
               w1lt_ref, b1l_ref, w1rt_ref, h1_ref):
  cnt = c0_ref[:, 0:1] + c1_ref[:, 0:1]
  inv = 1.0 / jnp.maximum(cnt, 1.0)
  mean = (a0_ref[...] + a1_ref[...]) * inv
  h1 = (jnp.dot(mean, w1lt_ref[...], preferred_element_type=jnp.float32)
        + b1l_ref[...]
        + jnp.dot(x_ref[...], w1rt_ref[...],
                  preferred_element_type=jnp.float32))
  h1_ref[...] = jnp.maximum(h1, 0.0)


def _tc_layer2(h1_ref, a0_ref, a1_ref, c0_ref, c1_ref,
               w2lt_ref, b2l_ref, w2rt_ref,
               wl1t_ref, wl2t_ref, blin_ref, out_ref):
  cnt = c0_ref[:, 0:1] + c1_ref[:, 0:1]
  inv = 1.0 / jnp.maximum(cnt, 1.0)
  mean = (a0_ref[...] + a1_ref[...]) * inv
  h1 = h1_ref[...]
  h2 = (jnp.dot(mean, w2lt_ref[...], preferred_element_type=jnp.float32)
        + b2l_ref[...]
        + jnp.dot(h1, w2rt_ref[...], preferred_element_type=jnp.float32))
  h2 = jnp.maximum(h2, 0.0)
  out = (jnp.dot(h1, wl1t_ref[...], preferred_element_type=jnp.float32)
         + jnp.dot(h2, wl2t_ref[...], preferred_element_type=jnp.float32)
         + blin_ref[...])
  out_ref[...] = jnp.maximum(out, 0.0)


def _row_spec(width):
  return pl.BlockSpec((RB, width), lambda i: (i, 0))


def _full_spec(shape):
  return pl.BlockSpec(shape, lambda i: tuple(0 for _ in shape))


def kernel(x, edge_index, W1l, b1l, W1r, W2l, b2l, W2r, Wlin, blin):
  src = jnp.concatenate(
      [edge_index[0].astype(jnp.int32),
       jnp.zeros((EPAD - E,), jnp.int32)]).reshape(NW, NCH, CH)
  dst = jnp.concatenate(
      [edge_index[1].astype(jnp.int32),
       jnp.full((EPAD - E,), PAD_DST, jnp.int32)]).reshape(NW, NCH, CH)
  zrow = jnp.zeros((RPT, D), jnp.float32)
  ones = jnp.ones((CH, D), jnp.float32)

  agg1, cnt = _sc_agg_cnt(x, src, dst, zrow, ones)
  c0 = cnt[0]
  c1 = cnt[1]

  grid = (N // RB,)
  h1 = pl.pallas_call(
      _tc_layer1,
      grid=grid,
      in_specs=[_row_spec(D), _row_spec(D), _row_spec(D),
                _row_spec(D), _row_spec(D),
                _full_spec((D, D)), _full_spec((D,)), _full_spec((D, D))],
      out_specs=_row_spec(D),
      out_shape=jax.ShapeDtypeStruct((N, D), jnp.float32),
  )(x, agg1[0], agg1[1], c0, c1, W1l.T, b1l, W1r.T)

  agg2 = _sc_agg(h1, src, dst, zrow)

  out = pl.pallas_call(
      _tc_layer2,
      grid=grid,
      in_specs=[_row_spec(D), _row_spec(D), _row_spec(D),
                _row_spec(D), _row_spec(D),
                _full_spec((D, D)), _full_spec((D,)), _full_spec((D, D)),
                _full_spec((D, D)), _full_spec((D, D)), _full_spec((D,))],
      out_specs=_row_spec(D),
      out_shape=jax.ShapeDtypeStruct((N, D), jnp.float32),
  )(h1, agg2[0], agg2[1], c0, c1, W2l.T, b2l, W2r.T,
    Wlin[:, :D].T, Wlin[:, D:].T, blin)
  return out


# speedup vs baseline: 1.4321x; 1.0371x over previous
import jax
import jax.numpy as jnp
from jax import lax
from jax.experimental import pallas as pl
from jax.experimental.pallas import tpu as pltpu
from jax.experimental.pallas import tpu_sc as plsc

N = 10000
E = 320000
D = 128

NC = 2
NS = 16
NW = NC * NS
CH = 128
NCH = 79
EPW = CH * NCH
EPAD = NW * EPW
NP = 10240
RPT = NP // NS
PAD_DST = NP - 8


_MESH = plsc.VectorSubcoreMesh(core_axis_name="c", subcore_axis_name="s")


def _make_sc_agg(with_cnt):
  def body(*refs):
    if with_cnt:
      (x_hbm, src_hbm, dst_hbm, zrow_hbm, ones_hbm,
       agg_out, cnt_out, src_v, dst_v, rows_v, agg_sh, sem) = refs
    else:
      (x_hbm, src_hbm, dst_hbm, zrow_hbm,
       agg_out, src_v, dst_v, rows_v, agg_sh, sem) = refs

    c = lax.axis_index("c")
    s = lax.axis_index("s")
    wid = c * NS + s
    base = s * RPT

    pltpu.sync_copy(src_hbm.at[wid], src_v)
    pltpu.sync_copy(dst_hbm.at[wid], dst_v)
    pltpu.sync_copy(zrow_hbm, agg_sh.at[pl.ds(base, RPT)])
    plsc.subcore_barrier()

    def chunk(j, carry):
      pltpu.async_copy(x_hbm.at[src_v.at[j]], rows_v, sem).wait()
      pltpu.sync_copy(rows_v, agg_sh.at[dst_v.at[j]], add=True)
      return carry

    lax.fori_loop(0, NCH, chunk, 0)
    plsc.subcore_barrier()

    pltpu.sync_copy(agg_sh.at[pl.ds(base, RPT)],
                    agg_out.at[c].at[pl.ds(base, RPT)])

    if with_cnt:
      pltpu.sync_copy(zrow_hbm, agg_sh.at[pl.ds(base, RPT)])
      pltpu.sync_copy(ones_hbm, rows_v)
      plsc.subcore_barrier()

      def chunk2(j, carry):
        pltpu.sync_copy(rows_v, agg_sh.at[dst_v.at[j]], add=True)
        return carry

      lax.fori_loop(0, NCH, chunk2, 0)
      plsc.subcore_barrier()
      pltpu.sync_copy(agg_sh.at[pl.ds(base, RPT)],
                      cnt_out.at[c].at[pl.ds(base, RPT)])

  out_type = [jax.ShapeDtypeStruct((NC, NP, D), jnp.float32)]
  if with_cnt:
    out_type.append(jax.ShapeDtypeStruct((NC, NP, D), jnp.float32))
  return pl.kernel(
      body,
      out_type=tuple(out_type) if with_cnt else out_type[0],
      mesh=_MESH,
      scratch_types=[
          pltpu.VMEM((NCH, CH), jnp.int32),
          pltpu.VMEM((NCH, CH), jnp.int32),
          pltpu.VMEM((CH, D), jnp.float32),
          pltpu.VMEM_SHARED((NP, D), jnp.float32),
          pltpu.SemaphoreType.DMA,
      ])


_sc_agg_cnt = _make_sc_agg(True)
_sc_agg = _make_sc_agg(False)

RB = 1000


def _tc_xw(x_ref, w_ref, xw_ref):
  xw_ref[...] = jnp.dot(x_ref[...], w_ref[...],
                        preferred_element_type=jnp.float32)


def _tc_layer1(xw_ref, a0_ref, a1_ref, c0_ref, c1_ref,
               w1lt_ref, b1l_ref, h1_ref):
  cnt = c0_ref[:, 0:1] + c1_ref[:, 0:1]
  inv = 1.0 / jnp.maximum(cnt, 1.0)
  mean = (a0_ref[...] + a1_ref[...]) * inv
  h1 = (jnp.dot(mean, w1lt_ref[...], preferred_element_type=jnp.float32)
        + b1l_ref[...] + xw_ref[...])
  h1_ref[...] = jnp.maximum(h1, 0.0)


def _tc_hw(h1_ref, w2rt_ref, wl1t_ref, hw_ref, hl_ref):
  h1 = h1_ref[...]
  hw_ref[...] = jnp.dot(h1, w2rt_ref[...], preferred_element_type=jnp.float32)
  hl_ref[...] = jnp.dot(h1, wl1t_ref[...], preferred_element_type=jnp.float32)


def _tc_layer2(hw_ref, hl_ref, a0_ref, a1_ref, c0_ref, c1_ref,
               w2lt_ref, b2l_ref, wl2t_ref, blin_ref, out_ref):
  cnt = c0_ref[:, 0:1] + c1_ref[:, 0:1]
  inv = 1.0 / jnp.maximum(cnt, 1.0)
  mean = (a0_ref[...] + a1_ref[...]) * inv
  h2 = (jnp.dot(mean, w2lt_ref[...], preferred_element_type=jnp.float32)
        + b2l_ref[...] + hw_ref[...])
  h2 = jnp.maximum(h2, 0.0)
  out = (jnp.dot(h2, wl2t_ref[...], preferred_element_type=jnp.float32)
         + hl_ref[...] + blin_ref[...])
  out_ref[...] = jnp.maximum(out, 0.0)


def _row_spec(width):
  return pl.BlockSpec((RB, width), lambda i: (i, 0))


def _full_spec(shape):
  return pl.BlockSpec(shape, lambda i: tuple(0 for _ in shape))


def kernel(x, edge_index, W1l, b1l, W1r, W2l, b2l, W2r, Wlin, blin):
  src = jnp.concatenate(
      [edge_index[0].astype(jnp.int32),
       jnp.zeros((EPAD - E,), jnp.int32)]).reshape(NW, NCH, CH)
  dst = jnp.concatenate(
      [edge_index[1].astype(jnp.int32),
       jnp.full((EPAD - E,), PAD_DST, jnp.int32)]).reshape(NW, NCH, CH)
  zrow = jnp.zeros((RPT, D), jnp.float32)
  ones = jnp.ones((CH, D), jnp.float32)

  grid = (N // RB,)
  agg1, cnt = _sc_agg_cnt(x, src, dst, zrow, ones)
  xw = pl.pallas_call(
      _tc_xw,
      grid=grid,
      in_specs=[_row_spec(D), _full_spec((D, D))],
      out_specs=_row_spec(D),
      out_shape=jax.ShapeDtypeStruct((N, D), jnp.float32),
  )(x, W1r.T)
  c0 = cnt[0]
  c1 = cnt[1]

  h1 = pl.pallas_call(
      _tc_layer1,
      grid=grid,
      in_specs=[_row_spec(D), _row_spec(D), _row_spec(D),
                _row_spec(D), _row_spec(D),
                _full_spec((D, D)), _full_spec((D,))],
      out_specs=_row_spec(D),
      out_shape=jax.ShapeDtypeStruct((N, D), jnp.float32),
  )(xw, agg1[0], agg1[1], c0, c1, W1l.T, b1l)

  agg2 = _sc_agg(h1, src, dst, zrow)
  hw, hl = pl.pallas_call(
      _tc_hw,
      grid=grid,
      in_specs=[_row_spec(D), _full_spec((D, D)), _full_spec((D, D))],
      out_specs=[_row_spec(D), _row_spec(D)],
      out_shape=[jax.ShapeDtypeStruct((N, D), jnp.float32),
                 jax.ShapeDtypeStruct((N, D), jnp.float32)],
  )(h1, W2r.T, Wlin[:, :D].T)

  out = pl.pallas_call(
      _tc_layer2,
      grid=grid,
      in_specs=[_row_spec(D), _row_spec(D), _row_spec(D), _row_spec(D),
                _row_spec(D), _row_spec(D),
                _full_spec((D, D)), _full_spec((D,)),
                _full_spec((D, D)), _full_spec((D,))],
      out_specs=_row_spec(D),
      out_shape=jax.ShapeDtypeStruct((N, D), jnp.float32),
  )(hw, hl, agg2[0], agg2[1], c0, c1, W2l.T, b2l, Wlin[:, D:].T, blin)
  return out
